# baseline (speedup 1.0000x reference)
import functools

import jax
import jax.numpy as jnp
from jax import lax
from jax.experimental import pallas as pl
from jax.experimental.pallas import tpu as pltpu
from jax.experimental.pallas import tpu_sc as plsc

N = 10000
NPAD = 10240
E = 160000
G = 64
NC = 2
NS = 16
NW = NC * NS
CH = 128
C0 = 60
C1 = 20
NCHROWS = NS * C0 + 15 * C1 + C0
EPAD = NCHROWS * CH
EPW = EPAD // NW
RPT = NPAD // NS
RB = 512
NBLK = NPAD // RB

_mesh = plsc.VectorSubcoreMesh(core_axis_name="c", subcore_axis_name="s")


@functools.partial(
    pl.kernel,
    out_type=jax.ShapeDtypeStruct((NW, NPAD), jnp.float32),
    mesh=_mesh,
    scratch_types=[
        pltpu.VMEM((EPW,), jnp.int32),
        pltpu.VMEM((NPAD,), jnp.float32),
    ],
    compiler_params=pltpu.CompilerParams(needs_layout_passes=False),
)
def _deg_kernel(dst_hbm, out_hbm, dstv, deg):
    cid = lax.axis_index("c")
    sid = lax.axis_index("s")
    wid = cid * NS + sid

    zeros16 = jnp.zeros((16,), jnp.float32)

    def zero_step(i, carry):
        deg[pl.ds(i * 16, 16)] = zeros16
        return carry

    lax.fori_loop(0, NPAD // 16, zero_step, 0)

    base = pl.multiple_of(wid * EPW, EPW)
    pltpu.sync_copy(dst_hbm.at[pl.ds(base, EPW)], dstv)

    ones16 = jnp.ones((16,), jnp.float32)

    def count_step(i, carry):
        idx = dstv[pl.ds(i * 16, 16)]
        plsc.addupdate_scatter(deg, [idx], ones16)
        return carry

    lax.fori_loop(0, EPW // 16, count_step, 0)
    pltpu.sync_copy(deg, out_hbm.at[wid])


def _make_agg(d_feat, nbuf, tc_tiling=True):
    @functools.partial(
        pl.kernel,
        out_type=jax.ShapeDtypeStruct((NC, NPAD, d_feat), jnp.float32),
        mesh=_mesh,
        scratch_types=[
            pltpu.VMEM((C0, CH), jnp.int32),
            pltpu.VMEM((C0, CH), jnp.int32),
            pltpu.VMEM((nbuf, CH, d_feat), jnp.float32),
            pltpu.VMEM_SHARED((NPAD, d_feat), jnp.float32),
        ] + [pltpu.SemaphoreType.DMA] * nbuf,
        compiler_params=pltpu.CompilerParams(use_tc_tiling_on_sc=tc_tiling),
    )
    def _agg_kernel(hs_hbm, src_hbm, dst_hbm, out_hbm, sv, dv, rows,
                    acc, *sems):
        cid = lax.axis_index("c")
        sid = lax.axis_index("s")
        nch = jnp.where(cid == 1, C0, C1)
        start = jnp.where(cid == 1, sid * C0, NS * C0 + sid * C1)

        pltpu.sync_copy(src_hbm.at[pl.ds(start, C0)], sv)
        pltpu.sync_copy(dst_hbm.at[pl.ds(start, C0)], dv)

        for b in range(nbuf):
            pltpu.async_copy(hs_hbm.at[sv.at[b]], rows.at[b], sems[b])

        r0 = pl.multiple_of(sid * RPT, RPT)
        pltpu.sync_copy(hs_hbm.at[pl.ds(r0, RPT)], acc.at[pl.ds(r0, RPT)])
        plsc.subcore_barrier()

        def group(g, carry):
            for b in range(nbuf):
                j = g * nbuf + b
                pltpu.make_async_copy(
                    hs_hbm.at[sv.at[j]], rows.at[b], sems[b]).wait()
                pltpu.sync_copy(rows.at[b], acc.at[dv.at[j]], add=True)

                @pl.when(j + nbuf < nch)
                def _():
                    pltpu.async_copy(
                        hs_hbm.at[sv.at[j + nbuf]], rows.at[b], sems[b])
            return carry

        lax.fori_loop(0, nch // nbuf, group, 0)
        plsc.subcore_barrier()
        pltpu.sync_copy(acc.at[pl.ds(r0, RPT)], out_hbm.at[cid, pl.ds(r0, RPT)])

    return _agg_kernel


_agg128 = _make_agg(128, 2, tc_tiling=False)
_agg64 = _make_agg(64, 4, tc_tiling=False)


def _mm1_body(x_ref, w_ref, degp_ref, h_ref):
    deg = jnp.sum(degp_ref[...], axis=0) + 1.0
    dinv = lax.rsqrt(deg)
    h = jnp.dot(x_ref[...], w_ref[...], preferred_element_type=jnp.float32)
    h_ref[...] = h * dinv[:, None]


def _mm1(xp, w1, degp):
    return pl.pallas_call(
        _mm1_body,
        grid=(NBLK,),
        in_specs=[
            pl.BlockSpec((RB, 384), lambda i: (i, 0)),
            pl.BlockSpec((384, 128), lambda i: (0, 0)),
            pl.BlockSpec((NW, RB), lambda i: (0, i)),
        ],
        out_specs=pl.BlockSpec((RB, 128), lambda i: (i, 0)),
        out_shape=jax.ShapeDtypeStruct((NPAD, 128), jnp.float32),
    )(xp, w1, degp)


def _mid_body(p0_ref, p1_ref, hs_ref, degp_ref, w_ref, b_ref, o_ref):
    deg = jnp.sum(degp_ref[...], axis=0) + 1.0
    dinv = lax.rsqrt(deg)
    agg = p0_ref[0] + p1_ref[0] - hs_ref[...]
    a = jnp.maximum(agg * dinv[:, None] + b_ref[...], 0.0)
    h = jnp.dot(a, w_ref[...], preferred_element_type=jnp.float32)
    o_ref[...] = h * dinv[:, None]


def _mid(p, hs1, degp, w2, b1):
    return pl.pallas_call(
        _mid_body,
        grid=(NBLK,),
        in_specs=[
            pl.BlockSpec((1, RB, 128), lambda i: (0, i, 0)),
            pl.BlockSpec((1, RB, 128), lambda i: (1, i, 0)),
            pl.BlockSpec((RB, 128), lambda i: (i, 0)),
            pl.BlockSpec((NW, RB), lambda i: (0, i)),
            pl.BlockSpec((128, 64), lambda i: (0, 0)),
            pl.BlockSpec((1, 128), lambda i: (0, 0)),
        ],
        out_specs=pl.BlockSpec((RB, 64), lambda i: (i, 0)),
        out_shape=jax.ShapeDtypeStruct((NPAD, 64), jnp.float32),
    )(p, p, hs1, degp, w2, b1)


def _pool_body(q0_ref, q1_ref, hs_ref, degp_ref, b_ref, batch_ref, wl_ref,
               bl_ref, out_ref, acc_ref):
    i = pl.program_id(0)
    deg = jnp.sum(degp_ref[...], axis=0) + 1.0
    dinv = lax.rsqrt(deg)
    agg = q0_ref[0] + q1_ref[0] - hs_ref[...]
    a = jnp.maximum(agg * dinv[:, None] + b_ref[...], 0.0)
    b = batch_ref[0, 0, :]
    onehot = (b[None, :] == lax.broadcasted_iota(jnp.int32, (G, RB), 0)
              ).astype(jnp.float32)
    aug = jnp.concatenate([a, jnp.ones((RB, 64), jnp.float32)], axis=1)
    part = jnp.dot(onehot, aug, preferred_element_type=jnp.float32)

    @pl.when(i == 0)
    def _():
        acc_ref[...] = part

    @pl.when(i > 0)
    def _():
        acc_ref[...] += part

    @pl.when(i == NBLK - 1)
    def _():
        pool = acc_ref[:, :64]
        cnt = jnp.maximum(acc_ref[:, 64:65], 1.0)
        mean = pool / cnt
        out_ref[...] = (
            jnp.dot(mean, wl_ref[...], preferred_element_type=jnp.float32)
            + bl_ref[...]
        )


def _pool(q, hs2, degp, b2, batchp, wl, bl):
    return pl.pallas_call(
        _pool_body,
        grid=(NBLK,),
        in_specs=[
            pl.BlockSpec((1, RB, 64), lambda i: (0, i, 0)),
            pl.BlockSpec((1, RB, 64), lambda i: (1, i, 0)),
            pl.BlockSpec((RB, 64), lambda i: (i, 0)),
            pl.BlockSpec((NW, RB), lambda i: (0, i)),
            pl.BlockSpec((1, 64), lambda i: (0, 0)),
            pl.BlockSpec((1, 1, RB), lambda i: (i, 0, 0)),
            pl.BlockSpec((64, 2), lambda i: (0, 0)),
            pl.BlockSpec((1, 2), lambda i: (0, 0)),
        ],
        out_specs=pl.BlockSpec((G, 2), lambda i: (0, 0)),
        out_shape=jax.ShapeDtypeStruct((G, 2), jnp.float32),
        scratch_shapes=[pltpu.VMEM((G, 128), jnp.float32)],
    )(q, q, hs2, degp, b2, batchp, wl, bl)


@jax.jit
def kernel(x, edge_index, batch, W1, b1, W2, b2, Wl, bl):
    src = edge_index[0].astype(jnp.int32)
    dst = edge_index[1].astype(jnp.int32)
    srcp = jnp.concatenate([src, jnp.zeros((EPAD - E,), jnp.int32)])
    dstp = jnp.concatenate([dst, jnp.full((EPAD - E,), N, jnp.int32)])
    srcc = srcp.reshape(NCHROWS, CH)
    dstc = dstp.reshape(NCHROWS, CH)
    xp = jnp.pad(x, ((0, NPAD - N), (0, 0)))
    batchp = jnp.concatenate(
        [batch.astype(jnp.int32), jnp.full((NPAD - N,), G, jnp.int32)]
    ).reshape(NBLK, 1, RB)

    degp = _deg_kernel(dstp)
    hs1 = _mm1(xp, W1, degp)
    p = _agg128(hs1, srcc, dstc)
    hs2 = _mid(p, hs1, degp, W2, b1.reshape(1, 128))
    q = _agg64(hs2, srcc, dstc)
    return _pool(q, hs2, degp, b2.reshape(1, 64), batchp,
                 Wl, bl.reshape(1, 2))


# speedup vs baseline: 1.3573x; 1.3573x over previous
import functools

import jax
import jax.numpy as jnp
from jax import lax
from jax.experimental import pallas as pl
from jax.experimental.pallas import tpu as pltpu
from jax.experimental.pallas import tpu_sc as plsc

N = 10000
NPAD = 10240
E = 160000
G = 64
NC = 2
NS = 16
NW = NC * NS
CH = 128
NCHROWS = 1280
CPW = NCHROWS // NS
EPAD = NCHROWS * CH
EPW = EPAD // NW
RPT = NPAD // NS
RB = 512
NBLK = NPAD // RB

_mesh = plsc.VectorSubcoreMesh(core_axis_name="c", subcore_axis_name="s")


@functools.partial(
    pl.kernel,
    out_type=jax.ShapeDtypeStruct((NW, NPAD), jnp.float32),
    mesh=_mesh,
    scratch_types=[
        pltpu.VMEM((EPW,), jnp.int32),
        pltpu.VMEM((NPAD,), jnp.float32),
    ],
    compiler_params=pltpu.CompilerParams(needs_layout_passes=False),
)
def _deg_kernel(dst_hbm, out_hbm, dstv, deg):
    cid = lax.axis_index("c")
    sid = lax.axis_index("s")
    wid = cid * NS + sid

    zeros16 = jnp.zeros((16,), jnp.float32)

    def zero_step(i, carry):
        deg[pl.ds(i * 16, 16)] = zeros16
        return carry

    lax.fori_loop(0, NPAD // 16, zero_step, 0)

    base = pl.multiple_of(wid * EPW, EPW)
    pltpu.sync_copy(dst_hbm.at[pl.ds(base, EPW)], dstv)

    ones16 = jnp.ones((16,), jnp.float32)

    def count_step(i, carry):
        idx = dstv[pl.ds(i * 16, 16)]
        plsc.addupdate_scatter(deg, [idx], ones16)
        return carry

    lax.fori_loop(0, EPW // 16, count_step, 0)
    pltpu.sync_copy(deg, out_hbm.at[wid])


def _make_agg(d_half, nbuf):
    @functools.partial(
        pl.kernel,
        out_type=jax.ShapeDtypeStruct((NC, NPAD, d_half), jnp.float32),
        mesh=_mesh,
        scratch_types=[
            pltpu.VMEM((CPW, CH), jnp.int32),
            pltpu.VMEM((CPW, CH), jnp.int32),
            pltpu.VMEM((nbuf, CH, d_half), jnp.float32),
            pltpu.VMEM_SHARED((NPAD, d_half), jnp.float32),
        ] + [pltpu.SemaphoreType.DMA] * nbuf,
        compiler_params=pltpu.CompilerParams(use_tc_tiling_on_sc=False),
    )
    def _agg_kernel(hs_hbm, src_hbm, dst_hbm, out_hbm, sv, dv, rows,
                    acc, *sems):
        cid = lax.axis_index("c")
        sid = lax.axis_index("s")
        start = pl.multiple_of(sid * CPW, CPW)

        pltpu.sync_copy(src_hbm.at[cid, pl.ds(start, CPW)], sv)
        pltpu.sync_copy(dst_hbm.at[pl.ds(start, CPW)], dv)

        for b in range(nbuf):
            pltpu.async_copy(hs_hbm.at[sv.at[b]], rows.at[b], sems[b])

        r0 = pl.multiple_of(sid * RPT, RPT)
        pltpu.sync_copy(hs_hbm.at[pl.ds(cid * NPAD + r0, RPT)],
                        acc.at[pl.ds(r0, RPT)])
        plsc.subcore_barrier()

        def group(g, carry):
            for b in range(nbuf):
                j = g * nbuf + b
                pltpu.make_async_copy(
                    hs_hbm.at[sv.at[j]], rows.at[b], sems[b]).wait()
                pltpu.sync_copy(rows.at[b], acc.at[dv.at[j]], add=True)

                @pl.when(j + nbuf < CPW)
                def _():
                    pltpu.async_copy(
                        hs_hbm.at[sv.at[j + nbuf]], rows.at[b], sems[b])
            return carry

        lax.fori_loop(0, CPW // nbuf, group, 0)
        plsc.subcore_barrier()
        pltpu.sync_copy(acc.at[pl.ds(r0, RPT)], out_hbm.at[cid, pl.ds(r0, RPT)])

    return _agg_kernel


_agg128 = _make_agg(64, 8)
_agg64 = _make_agg(32, 8)


def _mm1_body(x_ref, w_ref, degp_ref, h_ref):
    deg = jnp.sum(degp_ref[...], axis=0) + 1.0
    dinv = lax.rsqrt(deg)
    h = jnp.dot(x_ref[...], w_ref[...], preferred_element_type=jnp.float32)
    hs = h * dinv[:, None]
    h_ref[0] = hs[:, :64]
    h_ref[1] = hs[:, 64:]


def _mm1(xp, w1, degp):
    return pl.pallas_call(
        _mm1_body,
        grid=(NBLK,),
        in_specs=[
            pl.BlockSpec((RB, 384), lambda i: (i, 0)),
            pl.BlockSpec((384, 128), lambda i: (0, 0)),
            pl.BlockSpec((NW, RB), lambda i: (0, i)),
        ],
        out_specs=pl.BlockSpec((2, RB, 64), lambda i: (0, i, 0)),
        out_shape=jax.ShapeDtypeStruct((2, NPAD, 64), jnp.float32),
    )(xp, w1, degp)


def _mid_body(p0_ref, p1_ref, degp_ref, w_ref, b_ref, o_ref):
    deg = jnp.sum(degp_ref[...], axis=0) + 1.0
    dinv = lax.rsqrt(deg)
    agg = jnp.concatenate([p0_ref[0], p1_ref[0]], axis=1)
    a = jnp.maximum(agg * dinv[:, None] + b_ref[...], 0.0)
    h = jnp.dot(a, w_ref[...], preferred_element_type=jnp.float32)
    hs = h * dinv[:, None]
    o_ref[0] = hs[:, :32]
    o_ref[1] = hs[:, 32:]


def _mid(p, degp, w2, b1):
    return pl.pallas_call(
        _mid_body,
        grid=(NBLK,),
        in_specs=[
            pl.BlockSpec((1, RB, 64), lambda i: (0, i, 0)),
            pl.BlockSpec((1, RB, 64), lambda i: (1, i, 0)),
            pl.BlockSpec((NW, RB), lambda i: (0, i)),
            pl.BlockSpec((128, 64), lambda i: (0, 0)),
            pl.BlockSpec((1, 128), lambda i: (0, 0)),
        ],
        out_specs=pl.BlockSpec((2, RB, 32), lambda i: (0, i, 0)),
        out_shape=jax.ShapeDtypeStruct((2, NPAD, 32), jnp.float32),
    )(p, p, degp, w2, b1)


def _pool_body(q0_ref, q1_ref, degp_ref, b_ref, batch_ref, wl_ref,
               bl_ref, out_ref, acc_ref):
    i = pl.program_id(0)
    deg = jnp.sum(degp_ref[...], axis=0) + 1.0
    dinv = lax.rsqrt(deg)
    agg = jnp.concatenate([q0_ref[0], q1_ref[0]], axis=1)
    a = jnp.maximum(agg * dinv[:, None] + b_ref[...], 0.0)
    b = batch_ref[0, 0, :]
    onehot = (b[None, :] == lax.broadcasted_iota(jnp.int32, (G, RB), 0)
              ).astype(jnp.float32)
    aug = jnp.concatenate([a, jnp.ones((RB, 64), jnp.float32)], axis=1)
    part = jnp.dot(onehot, aug, preferred_element_type=jnp.float32)

    @pl.when(i == 0)
    def _():
        acc_ref[...] = part

    @pl.when(i > 0)
    def _():
        acc_ref[...] += part

    @pl.when(i == NBLK - 1)
    def _():
        pool = acc_ref[:, :64]
        cnt = jnp.maximum(acc_ref[:, 64:65], 1.0)
        mean = pool / cnt
        out_ref[...] = (
            jnp.dot(mean, wl_ref[...], preferred_element_type=jnp.float32)
            + bl_ref[...]
        )


def _pool(q, degp, b2, batchp, wl, bl):
    return pl.pallas_call(
        _pool_body,
        grid=(NBLK,),
        in_specs=[
            pl.BlockSpec((1, RB, 32), lambda i: (0, i, 0)),
            pl.BlockSpec((1, RB, 32), lambda i: (1, i, 0)),
            pl.BlockSpec((NW, RB), lambda i: (0, i)),
            pl.BlockSpec((1, 64), lambda i: (0, 0)),
            pl.BlockSpec((1, 1, RB), lambda i: (i, 0, 0)),
            pl.BlockSpec((64, 2), lambda i: (0, 0)),
            pl.BlockSpec((1, 2), lambda i: (0, 0)),
        ],
        out_specs=pl.BlockSpec((G, 2), lambda i: (0, 0)),
        out_shape=jax.ShapeDtypeStruct((G, 2), jnp.float32),
        scratch_shapes=[pltpu.VMEM((G, 128), jnp.float32)],
    )(q, q, degp, b2, batchp, wl, bl)


@jax.jit
def kernel(x, edge_index, batch, W1, b1, W2, b2, Wl, bl):
    src = edge_index[0].astype(jnp.int32)
    dst = edge_index[1].astype(jnp.int32)
    srcp = jnp.concatenate([src, jnp.zeros((EPAD - E,), jnp.int32)])
    dstp = jnp.concatenate([dst, jnp.full((EPAD - E,), N, jnp.int32)])
    srcc = srcp.reshape(NCHROWS, CH)
    src2 = jnp.stack([srcc, srcc + NPAD])
    dstc = dstp.reshape(NCHROWS, CH)
    xp = jnp.pad(x, ((0, NPAD - N), (0, 0)))
    batchp = jnp.concatenate(
        [batch.astype(jnp.int32), jnp.full((NPAD - N,), G, jnp.int32)]
    ).reshape(NBLK, 1, RB)

    degp = _deg_kernel(dstp)
    hs1 = _mm1(xp, W1, degp)
    p = _agg128(hs1.reshape(2 * NPAD, 64), src2, dstc)
    hs2 = _mid(p, degp, W2, b1.reshape(1, 128))
    q = _agg64(hs2.reshape(2 * NPAD, 32), src2, dstc)
    return _pool(q, degp, b2.reshape(1, 64), batchp,
                 Wl, bl.reshape(1, 2))


# speedup vs baseline: 1.3576x; 1.0002x over previous
import functools

import jax
import jax.numpy as jnp
from jax import lax
from jax.experimental import pallas as pl
from jax.experimental.pallas import tpu as pltpu
from jax.experimental.pallas import tpu_sc as plsc

N = 10000
NPAD = 10240
E = 160000
G = 64
NC = 2
NS = 16
NW = NC * NS
CH = 128
NCHROWS = 1280
CPW = NCHROWS // NS
EPAD = NCHROWS * CH
EPW = EPAD // NW
RPT = NPAD // NS
RB = 512
NBLK = NPAD // RB

_mesh = plsc.VectorSubcoreMesh(core_axis_name="c", subcore_axis_name="s")


@functools.partial(
    pl.kernel,
    out_type=jax.ShapeDtypeStruct((NW, NPAD), jnp.float32),
    mesh=_mesh,
    scratch_types=[
        pltpu.VMEM((EPW,), jnp.int32),
        pltpu.VMEM((NPAD,), jnp.float32),
    ],
    compiler_params=pltpu.CompilerParams(needs_layout_passes=False),
)
def _deg_kernel(dst_hbm, out_hbm, dstv, deg):
    cid = lax.axis_index("c")
    sid = lax.axis_index("s")
    wid = cid * NS + sid

    zeros16 = jnp.zeros((16,), jnp.float32)

    def zero_step(i, carry):
        deg[pl.ds(i * 16, 16)] = zeros16
        return carry

    lax.fori_loop(0, NPAD // 16, zero_step, 0)

    base = pl.multiple_of(wid * EPW, EPW)
    pltpu.sync_copy(dst_hbm.at[pl.ds(base, EPW)], dstv)

    ones16 = jnp.ones((16,), jnp.float32)

    def count_step(i, carry):
        idx = dstv[pl.ds(i * 16, 16)]
        plsc.addupdate_scatter(deg, [idx], ones16)
        return carry

    lax.fori_loop(0, EPW // 16, count_step, 0)
    pltpu.sync_copy(deg, out_hbm.at[wid])


def _make_agg(d_half, nbuf):
    @functools.partial(
        pl.kernel,
        out_type=jax.ShapeDtypeStruct((NC, NPAD, d_half), jnp.float32),
        mesh=_mesh,
        scratch_types=[
            pltpu.VMEM((CPW, CH), jnp.int32),
            pltpu.VMEM((CPW, CH), jnp.int32),
            pltpu.VMEM((nbuf, CH, d_half), jnp.float32),
            pltpu.VMEM_SHARED((NPAD, d_half), jnp.float32),
        ] + [pltpu.SemaphoreType.DMA] * nbuf,
        compiler_params=pltpu.CompilerParams(use_tc_tiling_on_sc=False),
    )
    def _agg_kernel(hs_hbm, src_hbm, dst_hbm, out_hbm, sv, dv, rows,
                    acc, *sems):
        cid = lax.axis_index("c")
        sid = lax.axis_index("s")
        start = pl.multiple_of(sid * CPW, CPW)

        pltpu.sync_copy(src_hbm.at[cid, pl.ds(start, CPW)], sv)
        pltpu.sync_copy(dst_hbm.at[pl.ds(start, CPW)], dv)

        for b in range(nbuf):
            pltpu.async_copy(hs_hbm.at[sv.at[b]], rows.at[b], sems[b])

        r0 = pl.multiple_of(sid * RPT, RPT)
        pltpu.sync_copy(hs_hbm.at[pl.ds(cid * NPAD + r0, RPT)],
                        acc.at[pl.ds(r0, RPT)])
        plsc.subcore_barrier()

        def group(g, carry):
            for b in range(nbuf):
                j = g * nbuf + b
                pltpu.make_async_copy(
                    hs_hbm.at[sv.at[j]], rows.at[b], sems[b]).wait()
                pltpu.sync_copy(rows.at[b], acc.at[dv.at[j]], add=True)

                @pl.when(j + nbuf < CPW)
                def _():
                    pltpu.async_copy(
                        hs_hbm.at[sv.at[j + nbuf]], rows.at[b], sems[b])
            return carry

        lax.fori_loop(0, CPW // nbuf, group, 0)
        plsc.subcore_barrier()
        pltpu.sync_copy(acc.at[pl.ds(r0, RPT)], out_hbm.at[cid, pl.ds(r0, RPT)])

    return _agg_kernel


_agg128 = _make_agg(64, 8)
_agg64 = _make_agg(32, 16)


def _mm1_body(x_ref, w_ref, degp_ref, h_ref):
    deg = jnp.sum(degp_ref[...], axis=0) + 1.0
    dinv = lax.rsqrt(deg)
    h = jnp.dot(x_ref[...], w_ref[...], preferred_element_type=jnp.float32)
    hs = h * dinv[:, None]
    h_ref[0] = hs[:, :64]
    h_ref[1] = hs[:, 64:]


def _mm1(xp, w1, degp):
    return pl.pallas_call(
        _mm1_body,
        grid=(NBLK,),
        in_specs=[
            pl.BlockSpec((RB, 384), lambda i: (i, 0)),
            pl.BlockSpec((384, 128), lambda i: (0, 0)),
            pl.BlockSpec((NW, RB), lambda i: (0, i)),
        ],
        out_specs=pl.BlockSpec((2, RB, 64), lambda i: (0, i, 0)),
        out_shape=jax.ShapeDtypeStruct((2, NPAD, 64), jnp.float32),
    )(xp, w1, degp)


def _mid_body(p0_ref, p1_ref, degp_ref, w_ref, b_ref, o_ref):
    deg = jnp.sum(degp_ref[...], axis=0) + 1.0
    dinv = lax.rsqrt(deg)
    agg = jnp.concatenate([p0_ref[0], p1_ref[0]], axis=1)
    a = jnp.maximum(agg * dinv[:, None] + b_ref[...], 0.0)
    h = jnp.dot(a, w_ref[...], preferred_element_type=jnp.float32)
    hs = h * dinv[:, None]
    o_ref[0] = hs[:, :32]
    o_ref[1] = hs[:, 32:]


def _mid(p, degp, w2, b1):
    return pl.pallas_call(
        _mid_body,
        grid=(NBLK,),
        in_specs=[
            pl.BlockSpec((1, RB, 64), lambda i: (0, i, 0)),
            pl.BlockSpec((1, RB, 64), lambda i: (1, i, 0)),
            pl.BlockSpec((NW, RB), lambda i: (0, i)),
            pl.BlockSpec((128, 64), lambda i: (0, 0)),
            pl.BlockSpec((1, 128), lambda i: (0, 0)),
        ],
        out_specs=pl.BlockSpec((2, RB, 32), lambda i: (0, i, 0)),
        out_shape=jax.ShapeDtypeStruct((2, NPAD, 32), jnp.float32),
    )(p, p, degp, w2, b1)


def _pool_body(q0_ref, q1_ref, degp_ref, b_ref, batch_ref, wl_ref,
               bl_ref, out_ref, acc_ref):
    i = pl.program_id(0)
    deg = jnp.sum(degp_ref[...], axis=0) + 1.0
    dinv = lax.rsqrt(deg)
    agg = jnp.concatenate([q0_ref[0], q1_ref[0]], axis=1)
    a = jnp.maximum(agg * dinv[:, None] + b_ref[...], 0.0)
    b = batch_ref[0, 0, :]
    onehot = (b[None, :] == lax.broadcasted_iota(jnp.int32, (G, RB), 0)
              ).astype(jnp.float32)
    aug = jnp.concatenate([a, jnp.ones((RB, 64), jnp.float32)], axis=1)
    part = jnp.dot(onehot, aug, preferred_element_type=jnp.float32)

    @pl.when(i == 0)
    def _():
        acc_ref[...] = part

    @pl.when(i > 0)
    def _():
        acc_ref[...] += part

    @pl.when(i == NBLK - 1)
    def _():
        pool = acc_ref[:, :64]
        cnt = jnp.maximum(acc_ref[:, 64:65], 1.0)
        mean = pool / cnt
        out_ref[...] = (
            jnp.dot(mean, wl_ref[...], preferred_element_type=jnp.float32)
            + bl_ref[...]
        )


def _pool(q, degp, b2, batchp, wl, bl):
    return pl.pallas_call(
        _pool_body,
        grid=(NBLK,),
        in_specs=[
            pl.BlockSpec((1, RB, 32), lambda i: (0, i, 0)),
            pl.BlockSpec((1, RB, 32), lambda i: (1, i, 0)),
            pl.BlockSpec((NW, RB), lambda i: (0, i)),
            pl.BlockSpec((1, 64), lambda i: (0, 0)),
            pl.BlockSpec((1, 1, RB), lambda i: (i, 0, 0)),
            pl.BlockSpec((64, 2), lambda i: (0, 0)),
            pl.BlockSpec((1, 2), lambda i: (0, 0)),
        ],
        out_specs=pl.BlockSpec((G, 2), lambda i: (0, 0)),
        out_shape=jax.ShapeDtypeStruct((G, 2), jnp.float32),
        scratch_shapes=[pltpu.VMEM((G, 128), jnp.float32)],
    )(q, q, degp, b2, batchp, wl, bl)


@jax.jit
def kernel(x, edge_index, batch, W1, b1, W2, b2, Wl, bl):
    src = edge_index[0].astype(jnp.int32)
    dst = edge_index[1].astype(jnp.int32)
    srcp = jnp.concatenate([src, jnp.zeros((EPAD - E,), jnp.int32)])
    dstp = jnp.concatenate([dst, jnp.full((EPAD - E,), N, jnp.int32)])
    srcc = srcp.reshape(NCHROWS, CH)
    src2 = jnp.stack([srcc, srcc + NPAD])
    dstc = dstp.reshape(NCHROWS, CH)
    xp = jnp.pad(x, ((0, NPAD - N), (0, 0)))
    batchp = jnp.concatenate(
        [batch.astype(jnp.int32), jnp.full((NPAD - N,), G, jnp.int32)]
    ).reshape(NBLK, 1, RB)

    degp = _deg_kernel(dstp)
    hs1 = _mm1(xp, W1, degp)
    p = _agg128(hs1.reshape(2 * NPAD, 64), src2, dstc)
    hs2 = _mid(p, degp, W2, b1.reshape(1, 128))
    q = _agg64(hs2.reshape(2 * NPAD, 32), src2, dstc)
    return _pool(q, degp, b2.reshape(1, 64), batchp,
                 Wl, bl.reshape(1, 2))
